# Initial kernel scaffold; baseline (speedup 1.0000x reference)
import jax
import jax.numpy as jnp
from jax.experimental import pallas as pl


def kernel(xs, tables):
    raise NotImplementedError("write your pallas kernel here")


# speedup vs baseline: 1.1630x; 1.1630x over previous
import functools

import jax
import jax.numpy as jnp
from jax import lax
from jax.experimental import pallas as pl
from jax.experimental.pallas import tpu as pltpu
from jax.experimental.pallas import tpu_sc as plsc

N_FIELDS = 26
VOCAB = 100000
WIDTH = 32
BATCH = 16384

NC = 2
NS = 16
NW = NC * NS
BPW = BATCH // NW
CHUNK = 128
NCHUNK = BPW // CHUNK
LANES = 16

_mesh = plsc.VectorSubcoreMesh(core_axis_name="c", subcore_axis_name="s")


@functools.partial(
    pl.kernel,
    out_type=jax.ShapeDtypeStruct((N_FIELDS * BATCH, WIDTH), jnp.float32),
    mesh=_mesh,
    scratch_types=[
        pltpu.VMEM((BPW,), jnp.int32),
        pltpu.VMEM((BPW, WIDTH), jnp.float32),
        pltpu.SemaphoreType.DMA,
    ],
    compiler_params=pltpu.CompilerParams(use_tc_tiling_on_sc=False),
)
def _embed_gather(xs_hbm, tab_hbm, out_hbm, idx_v, rows_v, sem):
    wid = lax.axis_index("s") * NC + lax.axis_index("c")
    base_b = wid * BPW

    def per_field(f, carry):
        pltpu.sync_copy(xs_hbm.at[f, pl.ds(base_b, BPW)], idx_v)
        off = f * VOCAB

        def add_off(i, c):
            sl = pl.ds(i * LANES, LANES)
            idx_v[sl] = idx_v[sl] + off
            return c

        lax.fori_loop(0, BPW // LANES, add_off, 0)

        for j in range(NCHUNK):
            pltpu.async_copy(
                tab_hbm.at[idx_v.at[pl.ds(j * CHUNK, CHUNK)]],
                rows_v.at[pl.ds(j * CHUNK, CHUNK)],
                sem,
            )
        for j in range(NCHUNK):
            pltpu.make_async_copy(
                tab_hbm.at[idx_v.at[pl.ds(j * CHUNK, CHUNK)]],
                rows_v.at[pl.ds(j * CHUNK, CHUNK)],
                sem,
            ).wait()

        pltpu.sync_copy(rows_v, out_hbm.at[pl.ds(f * BATCH + base_b, BPW)])
        return carry

    lax.fori_loop(0, N_FIELDS, per_field, 0)


def kernel(xs, tables):
    tab_flat = tables.reshape(N_FIELDS * VOCAB, WIDTH)
    out = _embed_gather(xs, tab_flat)
    return out.reshape(N_FIELDS, BATCH, WIDTH)


# speedup vs baseline: 1.1652x; 1.0019x over previous
import functools

import jax
import jax.numpy as jnp
from jax import lax
from jax.experimental import pallas as pl
from jax.experimental.pallas import tpu as pltpu
from jax.experimental.pallas import tpu_sc as plsc

N_FIELDS = 26
VOCAB = 100000
WIDTH = 32
BATCH = 16384

NC = 2
NS = 16
NW = NC * NS
BPW = BATCH // NW
CHUNK = 128
NCHUNK = BPW // CHUNK
LANES = 16

_mesh = plsc.VectorSubcoreMesh(core_axis_name="c", subcore_axis_name="s")


@functools.partial(
    pl.kernel,
    out_type=jax.ShapeDtypeStruct((N_FIELDS, BATCH, WIDTH), jnp.float32),
    mesh=_mesh,
    scratch_types=[
        pltpu.VMEM((BPW,), jnp.int32),
        pltpu.VMEM((BPW, WIDTH), jnp.float32),
        pltpu.SemaphoreType.DMA,
    ],
    compiler_params=pltpu.CompilerParams(use_tc_tiling_on_sc=False),
)
def _embed_gather(xs_hbm, tab_hbm, out_hbm, idx_v, rows_v, sem):
    wid = lax.axis_index("s") * NC + lax.axis_index("c")
    base_b = wid * BPW

    def per_field(f, carry):
        pltpu.sync_copy(xs_hbm.at[f, pl.ds(base_b, BPW)], idx_v)

        for j in range(NCHUNK):
            pltpu.async_copy(
                tab_hbm.at[f].at[idx_v.at[pl.ds(j * CHUNK, CHUNK)]],
                rows_v.at[pl.ds(j * CHUNK, CHUNK)],
                sem,
            )
        for j in range(NCHUNK):
            pltpu.make_async_copy(
                tab_hbm.at[f].at[idx_v.at[pl.ds(j * CHUNK, CHUNK)]],
                rows_v.at[pl.ds(j * CHUNK, CHUNK)],
                sem,
            ).wait()

        pltpu.sync_copy(rows_v, out_hbm.at[f, pl.ds(base_b, BPW)])
        return carry

    lax.fori_loop(0, N_FIELDS, per_field, 0)


def kernel(xs, tables):
    return _embed_gather(xs, tables)
